# Initial kernel scaffold; baseline (speedup 1.0000x reference)
import jax
import jax.numpy as jnp
from jax.experimental import pallas as pl


def kernel(x, edge_index, batch, W1, b1, W2, b2, Wfc, bfc):
    raise NotImplementedError("write your pallas kernel here")


# speedup vs baseline: 8.7063x; 8.7063x over previous
import functools

import jax
import jax.numpy as jnp
from jax import lax
from jax.experimental import pallas as pl
from jax.experimental.pallas import tpu as pltpu
from jax.experimental.pallas import tpu_sc as plsc

N = 10000
E = 320000
F_IN = 128
H = 256
G = 64
HALF = 128

NC = 2
NS = 16
K = 80

N_PAD = 10240
ROWS_PER_TILE = N_PAD // NS
ZROWS = 128

BN = 1000
NBLK = N // BN


def _leaky(v):
    return jnp.where(v > 0, v, 0.2 * v)


_EPT_DEG = E // (NC * NS)


@functools.partial(
    pl.kernel,
    mesh=plsc.VectorSubcoreMesh(core_axis_name="c", subcore_axis_name="s"),
    out_type=jax.ShapeDtypeStruct((NC, N_PAD, HALF), jnp.float32),
    scratch_types=[
        pltpu.VMEM((K,), jnp.int32),
        pltpu.VMEM((K, HALF), jnp.float32),
        pltpu.VMEM((ZROWS, HALF), jnp.float32),
        pltpu.VMEM_SHARED((N_PAD, HALF), jnp.float32),
        pltpu.SemaphoreType.DMA,
    ],
)
def _deg_kernel(dst_hbm, ones_hbm, zeros_hbm, out_hbm,
                idx_v, ones_v, zbuf_v, acc_sh, sem):
    c = lax.axis_index("c")
    s = lax.axis_index("s")
    w = s * NC + c

    pltpu.sync_copy(ones_hbm, ones_v)
    pltpu.sync_copy(zeros_hbm, zbuf_v)

    for z in range(5):
        pltpu.sync_copy(
            zbuf_v, acc_sh.at[pl.ds(s * ROWS_PER_TILE + z * ZROWS, ZROWS)]
        )
    plsc.subcore_barrier()

    base = w * _EPT_DEG

    def chunk(i, carry):
        pltpu.sync_copy(dst_hbm.at[pl.ds(base + i * K, K)], idx_v)
        pltpu.sync_copy(ones_v, acc_sh.at[idx_v], add=True)
        return carry

    lax.fori_loop(0, _EPT_DEG // K, chunk, 0)
    plsc.subcore_barrier()

    rslice = pl.ds(s * ROWS_PER_TILE, ROWS_PER_TILE)
    pltpu.sync_copy(acc_sh.at[rslice], out_hbm.at[c, rslice])


_EPT_AGG = E // NS


@functools.partial(
    pl.kernel,
    mesh=plsc.VectorSubcoreMesh(core_axis_name="c", subcore_axis_name="s"),
    out_type=jax.ShapeDtypeStruct((NC, N_PAD, HALF), jnp.float32),
    scratch_types=[
        pltpu.VMEM((K,), jnp.int32),
        pltpu.VMEM((K,), jnp.int32),
        pltpu.VMEM((K, HALF), jnp.float32),
        pltpu.VMEM((ZROWS, HALF), jnp.float32),
        pltpu.VMEM_SHARED((N_PAD, HALF), jnp.float32),
        pltpu.SemaphoreType.DMA,
    ],
)
def _agg_kernel(hp_hbm, src_hbm, dst_hbm, out_hbm,
                srcv, dstv, rows_v, zbuf_v, acc_sh, sem):
    c = lax.axis_index("c")
    s = lax.axis_index("s")

    def fz(i, carry):
        r = i // 8
        k = i % 8
        zbuf_v[r, pl.ds(k * 16, 16)] = jnp.zeros((16,), jnp.float32)
        return carry

    lax.fori_loop(0, ZROWS * 8, fz, 0)

    for z in range(5):
        pltpu.sync_copy(
            zbuf_v, acc_sh.at[pl.ds(s * ROWS_PER_TILE + z * ZROWS, ZROWS)]
        )
    plsc.subcore_barrier()

    base = s * _EPT_AGG
    coff = c * N

    def chunk(i, carry):
        off = base + i * K
        pltpu.sync_copy(src_hbm.at[pl.ds(off, K)], srcv)
        pltpu.sync_copy(dst_hbm.at[pl.ds(off, K)], dstv)
        for v in range(K // 16):
            srcv[pl.ds(v * 16, 16)] = srcv[pl.ds(v * 16, 16)] + coff
        pltpu.async_copy(hp_hbm.at[srcv], rows_v, sem).wait()
        pltpu.sync_copy(rows_v, acc_sh.at[dstv], add=True)
        return carry

    lax.fori_loop(0, _EPT_AGG // K, chunk, 0)
    plsc.subcore_barrier()

    rslice = pl.ds(s * ROWS_PER_TILE, ROWS_PER_TILE)
    pltpu.sync_copy(acc_sh.at[rslice], out_hbm.at[c, rslice])


def _tc1_body(x_ref, w1_ref, degp_ref, hp_ref):
    deg = degp_ref[0][:, 0:1] + degp_ref[1][:, 0:1] + 1.0
    dis = lax.rsqrt(deg)
    h = jnp.dot(x_ref[...], w1_ref[...], preferred_element_type=jnp.float32)
    hp = h * dis
    hp_ref[0] = hp[:, :HALF]
    hp_ref[1] = hp[:, HALF:]


_tc1 = pl.pallas_call(
    _tc1_body,
    grid=(NBLK,),
    in_specs=[
        pl.BlockSpec((BN, F_IN), lambda i: (i, 0)),
        pl.BlockSpec((F_IN, H), lambda i: (0, 0)),
        pl.BlockSpec((NC, BN, HALF), lambda i: (0, i, 0)),
    ],
    out_specs=pl.BlockSpec((NC, BN, HALF), lambda i: (0, i, 0)),
    out_shape=jax.ShapeDtypeStruct((NC, N, HALF), jnp.float32),
)


def _tc2_body(agg_ref, hp_ref, degp_ref, b1_ref, w2_ref, hq_ref):
    deg = degp_ref[0][:, 0:1] + degp_ref[1][:, 0:1] + 1.0
    dis = lax.rsqrt(deg)
    b1 = b1_ref[...]
    h1a = _leaky(dis * (agg_ref[0] + hp_ref[0]) + b1[:, :HALF])
    h1b = _leaky(dis * (agg_ref[1] + hp_ref[1]) + b1[:, HALF:])
    w2 = w2_ref[...]
    h2 = (jnp.dot(h1a, w2[:HALF, :], preferred_element_type=jnp.float32)
          + jnp.dot(h1b, w2[HALF:, :], preferred_element_type=jnp.float32))
    hq = h2 * dis
    hq_ref[0] = hq[:, :HALF]
    hq_ref[1] = hq[:, HALF:]


_tc2 = pl.pallas_call(
    _tc2_body,
    grid=(NBLK,),
    in_specs=[
        pl.BlockSpec((NC, BN, HALF), lambda i: (0, i, 0)),
        pl.BlockSpec((NC, BN, HALF), lambda i: (0, i, 0)),
        pl.BlockSpec((NC, BN, HALF), lambda i: (0, i, 0)),
        pl.BlockSpec((1, H), lambda i: (0, 0)),
        pl.BlockSpec((H, H), lambda i: (0, 0)),
    ],
    out_specs=pl.BlockSpec((NC, BN, HALF), lambda i: (0, i, 0)),
    out_shape=jax.ShapeDtypeStruct((NC, N, HALF), jnp.float32),
)


def _tc3_body(agg_ref, hq_ref, degp_ref, b2_ref,
              batch_ref, wfc_ref, bfc_ref, out_ref, sums, cnts):
    i = pl.program_id(0)

    @pl.when(i == 0)
    def _init():
        sums[...] = jnp.zeros_like(sums)
        cnts[...] = jnp.zeros_like(cnts)

    deg = degp_ref[0][:, 0:1] + degp_ref[1][:, 0:1] + 1.0
    dis = lax.rsqrt(deg)
    b2 = b2_ref[...]
    h2a = _leaky(dis * (agg_ref[0] + hq_ref[0]) + b2[:, :HALF])
    h2b = _leaky(dis * (agg_ref[1] + hq_ref[1]) + b2[:, HALF:])
    h2 = jnp.concatenate([h2a, h2b], axis=1)

    bb = batch_ref[...]
    mgT = (bb == lax.broadcasted_iota(jnp.int32, (BN, G), 1)).astype(
        jnp.float32)
    sums[...] += lax.dot_general(
        mgT, h2, (((0,), (0,)), ((), ())), preferred_element_type=jnp.float32)
    cnts[...] += lax.dot_general(
        mgT, jnp.ones((BN, 128), jnp.float32), (((0,), (0,)), ((), ())),
        preferred_element_type=jnp.float32)

    @pl.when(i == NBLK - 1)
    def _fin():
        cnt = jnp.maximum(cnts[:, 0:1], 1.0)
        pooled = sums[...] / cnt
        z = jnp.dot(pooled, wfc_ref[...], preferred_element_type=jnp.float32)
        zb = z + bfc_ref[...]
        out_ref[...] = 1.0 / (1.0 + jnp.exp(-zb))


_tc3 = pl.pallas_call(
    _tc3_body,
    grid=(NBLK,),
    in_specs=[
        pl.BlockSpec((NC, BN, HALF), lambda i: (0, i, 0)),
        pl.BlockSpec((NC, BN, HALF), lambda i: (0, i, 0)),
        pl.BlockSpec((NC, BN, HALF), lambda i: (0, i, 0)),
        pl.BlockSpec((1, H), lambda i: (0, 0)),
        pl.BlockSpec((BN, 1), lambda i: (i, 0)),
        pl.BlockSpec((H, 128), lambda i: (0, 0)),
        pl.BlockSpec((1, 128), lambda i: (0, 0)),
    ],
    out_specs=pl.BlockSpec((G, 128), lambda i: (0, 0)),
    out_shape=jax.ShapeDtypeStruct((G, 128), jnp.float32),
    scratch_shapes=[
        pltpu.VMEM((G, H), jnp.float32),
        pltpu.VMEM((G, 128), jnp.float32),
    ],
)


def kernel(x, edge_index, batch, W1, b1, W2, b2, Wfc, bfc):
    src = edge_index[0]
    dst = edge_index[1]

    onesk = jnp.ones((K, HALF), jnp.float32)
    zerosk = jnp.zeros((ZROWS, HALF), jnp.float32)
    degp = _deg_kernel(dst, onesk, zerosk)
    hp = _tc1(x, W1, degp)
    agg = _agg_kernel(hp.reshape(NC * N, HALF), src, dst)
    hq = _tc2(agg, hp, degp, b1.reshape(1, H), W2)
    aggb = _agg_kernel(hq.reshape(NC * N, HALF), src, dst)

    wfc_pad = jnp.pad(Wfc, ((0, 0), (0, 127)))
    bfc_pad = jnp.broadcast_to(bfc.reshape(1, 1), (1, 128))
    out = _tc3(aggb, hq, degp, b2.reshape(1, H),
               batch.reshape(N, 1), wfc_pad, bfc_pad)
    return out[:, 0]


# speedup vs baseline: 20.7445x; 2.3827x over previous
import functools

import jax
import jax.numpy as jnp
from jax import lax
from jax.experimental import pallas as pl
from jax.experimental.pallas import tpu as pltpu
from jax.experimental.pallas import tpu_sc as plsc

N = 10000
E = 320000
F_IN = 128
H = 256
G = 64
HALF = 128

NC = 2
NS = 16
K = 128
E_PAD = 327680

N_PAD = 10240
ROWS_PER_TILE = N_PAD // NS
ZROWS = 128

BN = 1000
NBLK = N // BN


def _leaky(v):
    return jnp.where(v > 0, v, 0.2 * v)


_EPT_DEG = E_PAD // (NC * NS)


_NCH_DEG = _EPT_DEG // K


@functools.partial(
    pl.kernel,
    mesh=plsc.VectorSubcoreMesh(core_axis_name="c", subcore_axis_name="s"),
    out_type=jax.ShapeDtypeStruct((NC, N_PAD, HALF), jnp.float32),
    scratch_types=[
        pltpu.VMEM((_EPT_DEG,), jnp.int32),
        pltpu.VMEM((K,), jnp.int32),
        pltpu.VMEM((K,), jnp.int32),
        pltpu.VMEM((K, HALF), jnp.float32),
        pltpu.VMEM((ZROWS, HALF), jnp.float32),
        pltpu.VMEM_SHARED((N_PAD, HALF), jnp.float32),
        pltpu.SemaphoreType.DMA,
    ],
)
def _deg_kernel(dst_hbm, ones_hbm, zeros_hbm, out_hbm,
                dst_all, dstv_a, dstv_b, ones_v, zbuf_v, acc_sh, sem):
    c = lax.axis_index("c")
    s = lax.axis_index("s")
    w = s * NC + c

    pltpu.sync_copy(dst_hbm.at[pl.ds(w * _EPT_DEG, _EPT_DEG)], dst_all)
    pltpu.sync_copy(ones_hbm, ones_v)
    pltpu.sync_copy(zeros_hbm, zbuf_v)

    for z in range(5):
        pltpu.sync_copy(
            zbuf_v, acc_sh.at[pl.ds(s * ROWS_PER_TILE + z * ZROWS, ZROWS)]
        )
    plsc.subcore_barrier()

    def idx_copy(i, dv):
        for v in range(K // 16):
            dv[pl.ds(v * 16, 16)] = dst_all[pl.ds(i * K + v * 16, 16)]

    def pair(j, carry):
        i0 = 2 * j
        i1 = i0 + 1
        idx_copy(i0, dstv_a)
        pltpu.async_copy(ones_v, acc_sh.at[dstv_a], sem, add=True)
        idx_copy(i1, dstv_b)
        pltpu.async_copy(ones_v, acc_sh.at[dstv_b], sem, add=True)
        pltpu.make_async_copy(ones_v, acc_sh.at[dstv_a], sem).wait()
        pltpu.make_async_copy(ones_v, acc_sh.at[dstv_b], sem).wait()
        return carry

    lax.fori_loop(0, _NCH_DEG // 2, pair, 0)
    plsc.subcore_barrier()

    for z in range(5):
        zsl = pl.ds(s * ROWS_PER_TILE + z * ZROWS, ZROWS)
        pltpu.sync_copy(acc_sh.at[zsl], zbuf_v)
        pltpu.sync_copy(zbuf_v, out_hbm.at[c, zsl])


_EPT_AGG = E_PAD // NS


_NPHASE = 10
_EPP = _EPT_AGG // _NPHASE
_NCHP = _EPP // K
_NHP = _NCHP // 2


@functools.partial(
    pl.kernel,
    mesh=plsc.VectorSubcoreMesh(core_axis_name="c", subcore_axis_name="s"),
    out_type=jax.ShapeDtypeStruct((NC, N_PAD, HALF), jnp.float32),
    scratch_types=[
        pltpu.VMEM((_EPP,), jnp.int32),
        pltpu.VMEM((_EPP,), jnp.int32),
        pltpu.VMEM((_EPP,), jnp.int32),
        pltpu.VMEM((_EPP,), jnp.int32),
        pltpu.VMEM((K,), jnp.int32),
        pltpu.VMEM((K,), jnp.int32),
        pltpu.VMEM((K, HALF), jnp.float32),
        pltpu.VMEM((K, HALF), jnp.float32),
        pltpu.VMEM_SHARED((N_PAD, HALF), jnp.float32),
        pltpu.SemaphoreType.DMA,
        pltpu.SemaphoreType.DMA,
        pltpu.SemaphoreType.DMA,
        pltpu.SemaphoreType.DMA,
        pltpu.SemaphoreType.DMA,
    ],
)
def _agg_kernel(hp_hbm, src_hbm, dst_hbm, zeros_hbm, out_hbm,
                src_a, src_b, dst_a, dst_b, dstv_a, dstv_b,
                rows_a, rows_b, acc_sh,
                sem_a, sem_b, sem_sa, sem_sb, sem_i):
    c = lax.axis_index("c")
    s = lax.axis_index("s")

    base = s * _EPT_AGG
    coff = c * N

    def adjust(src_buf):
        def fadj(i, carry):
            sl = pl.ds(i * 16, 16)
            src_buf[sl] = src_buf[sl] + coff
            return carry
        lax.fori_loop(0, _EPP // 16, fadj, 0)

    pltpu.sync_copy(src_hbm.at[pl.ds(base, _EPP)], src_a)
    pltpu.sync_copy(dst_hbm.at[pl.ds(base, _EPP)], dst_a)
    adjust(src_a)

    pltpu.sync_copy(zeros_hbm.at[pl.ds(0, K)], rows_a)
    for z in range(ROWS_PER_TILE // K):
        pltpu.sync_copy(
            rows_a, acc_sh.at[pl.ds(s * ROWS_PER_TILE + z * K, K)]
        )
    plsc.subcore_barrier()

    def g_start(src_buf, i, rows, sem):
        pltpu.make_async_copy(hp_hbm.at[src_buf.at[pl.ds(i * K, K)]],
                              rows, sem).start()

    def g_wait(src_buf, i, rows, sem):
        pltpu.make_async_copy(hp_hbm.at[src_buf.at[pl.ds(i * K, K)]],
                              rows, sem).wait()

    def idx_copy(dst_buf, i, dv):
        for v in range(K // 16):
            dv[pl.ds(v * 16, 16)] = dst_buf[pl.ds(i * K + v * 16, 16)]

    def s_start(rows, dv, sem):
        pltpu.async_copy(rows, acc_sh.at[dv], sem, add=True)

    def s_wait(rows, dv, sem):
        pltpu.make_async_copy(rows, acc_sh.at[dv], sem).wait()

    bufs = [(src_a, dst_a), (src_b, dst_b)]
    for p in range(_NPHASE):
        src_c, dst_c = bufs[p % 2]
        src_n, dst_n = bufs[(p + 1) % 2]
        if p + 1 < _NPHASE:
            noff = base + (p + 1) * _EPP
            pltpu.make_async_copy(
                src_hbm.at[pl.ds(noff, _EPP)], src_n, sem_i).start()
            pltpu.make_async_copy(
                dst_hbm.at[pl.ds(noff, _EPP)], dst_n, sem_i).start()

        g_start(src_c, 0, rows_a, sem_a)
        g_start(src_c, 1, rows_b, sem_b)

        def body(j, carry):
            i0 = 2 * j
            i1 = i0 + 1
            g_wait(src_c, i0, rows_a, sem_a)
            idx_copy(dst_c, i0, dstv_a)
            pltpu.sync_copy(rows_a, acc_sh.at[dstv_a], add=True)

            @pl.when(j < _NHP - 1)
            def _prea():
                g_start(src_c, i0 + 2, rows_a, sem_a)

            g_wait(src_c, i1, rows_b, sem_b)
            idx_copy(dst_c, i1, dstv_b)
            pltpu.sync_copy(rows_b, acc_sh.at[dstv_b], add=True)

            @pl.when(j < _NHP - 1)
            def _preb():
                g_start(src_c, i1 + 2, rows_b, sem_b)

            return carry

        lax.fori_loop(0, _NHP, body, 0)

        if p + 1 < _NPHASE:
            pltpu.make_async_copy(
                src_hbm.at[pl.ds(noff, _EPP)], src_n, sem_i).wait()
            pltpu.make_async_copy(
                dst_hbm.at[pl.ds(noff, _EPP)], dst_n, sem_i).wait()
            adjust(src_n)

    plsc.subcore_barrier()

    for z in range(ROWS_PER_TILE // K):
        zsl = pl.ds(s * ROWS_PER_TILE + z * K, K)
        pltpu.sync_copy(acc_sh.at[zsl], rows_a)
        pltpu.sync_copy(rows_a, out_hbm.at[c, zsl])


def _tc1_body(x_ref, w1_ref, degp_ref, hp_ref):
    deg = degp_ref[0] + degp_ref[1] + 1.0
    dis = lax.rsqrt(deg)
    h = jnp.dot(x_ref[...], w1_ref[...], preferred_element_type=jnp.float32)
    hp = h * dis
    hp_ref[0] = hp[:, :HALF]
    hp_ref[1] = hp[:, HALF:]


_tc1 = pl.pallas_call(
    _tc1_body,
    grid=(NBLK,),
    in_specs=[
        pl.BlockSpec((BN, F_IN), lambda i: (i, 0)),
        pl.BlockSpec((F_IN, H), lambda i: (0, 0)),
        pl.BlockSpec((NC, BN, 1), lambda i: (0, i, 0)),
    ],
    out_specs=pl.BlockSpec((NC, BN, HALF), lambda i: (0, i, 0)),
    out_shape=jax.ShapeDtypeStruct((NC, N, HALF), jnp.float32),
)


def _tc2_body(agg_ref, hp_ref, degp_ref, b1_ref, w2_ref, hq_ref):
    deg = degp_ref[0] + degp_ref[1] + 1.0
    dis = lax.rsqrt(deg)
    b1 = b1_ref[...]
    h1a = _leaky(dis * (agg_ref[0] + hp_ref[0]) + b1[:, :HALF])
    h1b = _leaky(dis * (agg_ref[1] + hp_ref[1]) + b1[:, HALF:])
    w2 = w2_ref[...]
    h2 = (jnp.dot(h1a, w2[:HALF, :], preferred_element_type=jnp.float32)
          + jnp.dot(h1b, w2[HALF:, :], preferred_element_type=jnp.float32))
    hq = h2 * dis
    hq_ref[0] = hq[:, :HALF]
    hq_ref[1] = hq[:, HALF:]


_tc2 = pl.pallas_call(
    _tc2_body,
    grid=(NBLK,),
    in_specs=[
        pl.BlockSpec((NC, BN, HALF), lambda i: (0, i, 0)),
        pl.BlockSpec((NC, BN, HALF), lambda i: (0, i, 0)),
        pl.BlockSpec((NC, BN, 1), lambda i: (0, i, 0)),
        pl.BlockSpec((1, H), lambda i: (0, 0)),
        pl.BlockSpec((H, H), lambda i: (0, 0)),
    ],
    out_specs=pl.BlockSpec((NC, BN, HALF), lambda i: (0, i, 0)),
    out_shape=jax.ShapeDtypeStruct((NC, N, HALF), jnp.float32),
)


def _tc3_body(agg_ref, hq_ref, degp_ref, b2_ref,
              batch_ref, wfc_ref, bfc_ref, out_ref, sums, cnts):
    i = pl.program_id(0)

    @pl.when(i == 0)
    def _init():
        sums[...] = jnp.zeros_like(sums)
        cnts[...] = jnp.zeros_like(cnts)

    deg = degp_ref[0] + degp_ref[1] + 1.0
    dis = lax.rsqrt(deg)
    b2 = b2_ref[...]
    h2a = _leaky(dis * (agg_ref[0] + hq_ref[0]) + b2[:, :HALF])
    h2b = _leaky(dis * (agg_ref[1] + hq_ref[1]) + b2[:, HALF:])
    h2 = jnp.concatenate([h2a, h2b], axis=1)

    bb = batch_ref[...]
    mgT = (bb == lax.broadcasted_iota(jnp.int32, (BN, G), 1)).astype(
        jnp.float32)
    sums[...] += lax.dot_general(
        mgT, h2, (((0,), (0,)), ((), ())), preferred_element_type=jnp.float32)
    cnts[...] += lax.dot_general(
        mgT, jnp.ones((BN, 128), jnp.float32), (((0,), (0,)), ((), ())),
        preferred_element_type=jnp.float32)

    @pl.when(i == NBLK - 1)
    def _fin():
        cnt = jnp.maximum(cnts[:, 0:1], 1.0)
        pooled = sums[...] / cnt
        z = jnp.dot(pooled, wfc_ref[...], preferred_element_type=jnp.float32)
        zb = z + bfc_ref[...]
        out_ref[...] = 1.0 / (1.0 + jnp.exp(-zb))


_tc3 = pl.pallas_call(
    _tc3_body,
    grid=(NBLK,),
    in_specs=[
        pl.BlockSpec((NC, BN, HALF), lambda i: (0, i, 0)),
        pl.BlockSpec((NC, BN, HALF), lambda i: (0, i, 0)),
        pl.BlockSpec((NC, BN, 1), lambda i: (0, i, 0)),
        pl.BlockSpec((1, H), lambda i: (0, 0)),
        pl.BlockSpec((BN, 1), lambda i: (i, 0)),
        pl.BlockSpec((H, 128), lambda i: (0, 0)),
        pl.BlockSpec((1, 128), lambda i: (0, 0)),
    ],
    out_specs=pl.BlockSpec((G, 128), lambda i: (0, 0)),
    out_shape=jax.ShapeDtypeStruct((G, 128), jnp.float32),
    scratch_shapes=[
        pltpu.VMEM((G, H), jnp.float32),
        pltpu.VMEM((G, 128), jnp.float32),
    ],
)


def kernel(x, edge_index, batch, W1, b1, W2, b2, Wfc, bfc):
    npad = E_PAD - E
    pad_iota = jnp.arange(npad, dtype=jnp.int32)
    src = jnp.concatenate([edge_index[0], pad_iota % N])
    dst = jnp.concatenate([edge_index[1], N + pad_iota % (N_PAD - N)])

    onesk = jnp.ones((K, HALF), jnp.float32)
    zerosk = jnp.zeros((ZROWS, HALF), jnp.float32)
    degp = _deg_kernel(dst, onesk, zerosk)[:, :, 0:1]
    hp = _tc1(x, W1, degp)
    agg = _agg_kernel(hp.reshape(NC * N, HALF), src, dst, zerosk)
    hq = _tc2(agg, hp, degp, b1.reshape(1, H), W2)
    aggb = _agg_kernel(hq.reshape(NC * N, HALF), src, dst, zerosk)

    wfc_pad = jnp.pad(Wfc, ((0, 0), (0, 127)))
    bfc_pad = jnp.broadcast_to(bfc.reshape(1, 1), (1, 128))
    out = _tc3(aggb, hq, degp, b2.reshape(1, H),
               batch.reshape(N, 1), wfc_pad, bfc_pad)
    return out[:, 0]


# speedup vs baseline: 20.9669x; 1.0107x over previous
import functools

import jax
import jax.numpy as jnp
from jax import lax
from jax.experimental import pallas as pl
from jax.experimental.pallas import tpu as pltpu
from jax.experimental.pallas import tpu_sc as plsc

N = 10000
E = 320000
F_IN = 128
H = 256
G = 64
HALF = 128

NC = 2
NS = 16
K = 128
E_PAD = 327680

N_PAD = 10240
ROWS_PER_TILE = N_PAD // NS
ZROWS = 128

BN = 2000
NBLK = N // BN


def _leaky(v):
    return jnp.where(v > 0, v, 0.2 * v)


_EPT_DEG = E_PAD // (NC * NS)


_NCH_DEG = _EPT_DEG // K


@functools.partial(
    pl.kernel,
    mesh=plsc.VectorSubcoreMesh(core_axis_name="c", subcore_axis_name="s"),
    out_type=jax.ShapeDtypeStruct((NC, N_PAD, HALF), jnp.float32),
    scratch_types=[
        pltpu.VMEM((_EPT_DEG,), jnp.int32),
        pltpu.VMEM((K,), jnp.int32),
        pltpu.VMEM((K,), jnp.int32),
        pltpu.VMEM((K, HALF), jnp.float32),
        pltpu.VMEM((ZROWS, HALF), jnp.float32),
        pltpu.VMEM_SHARED((N_PAD, HALF), jnp.float32),
        pltpu.SemaphoreType.DMA,
    ],
)
def _deg_kernel(dst_hbm, ones_hbm, zeros_hbm, out_hbm,
                dst_all, dstv_a, dstv_b, ones_v, zbuf_v, acc_sh, sem):
    c = lax.axis_index("c")
    s = lax.axis_index("s")
    w = s * NC + c

    pltpu.sync_copy(dst_hbm.at[pl.ds(w * _EPT_DEG, _EPT_DEG)], dst_all)
    pltpu.sync_copy(ones_hbm, ones_v)
    pltpu.sync_copy(zeros_hbm, zbuf_v)

    for z in range(5):
        pltpu.sync_copy(
            zbuf_v, acc_sh.at[pl.ds(s * ROWS_PER_TILE + z * ZROWS, ZROWS)]
        )
    plsc.subcore_barrier()

    def idx_copy(i, dv):
        for v in range(K // 16):
            dv[pl.ds(v * 16, 16)] = dst_all[pl.ds(i * K + v * 16, 16)]

    def pair(j, carry):
        i0 = 2 * j
        i1 = i0 + 1
        idx_copy(i0, dstv_a)
        pltpu.async_copy(ones_v, acc_sh.at[dstv_a], sem, add=True)
        idx_copy(i1, dstv_b)
        pltpu.async_copy(ones_v, acc_sh.at[dstv_b], sem, add=True)
        pltpu.make_async_copy(ones_v, acc_sh.at[dstv_a], sem).wait()
        pltpu.make_async_copy(ones_v, acc_sh.at[dstv_b], sem).wait()
        return carry

    lax.fori_loop(0, _NCH_DEG // 2, pair, 0)
    plsc.subcore_barrier()

    for z in range(5):
        zsl = pl.ds(s * ROWS_PER_TILE + z * ZROWS, ZROWS)
        pltpu.sync_copy(acc_sh.at[zsl], zbuf_v)
        pltpu.sync_copy(zbuf_v, out_hbm.at[c, zsl])


_EPT_AGG = E_PAD // NS


_NPHASE = 10
_EPP = _EPT_AGG // _NPHASE
_NCHP = _EPP // K
_NHP = _NCHP // 2


@functools.partial(
    pl.kernel,
    mesh=plsc.VectorSubcoreMesh(core_axis_name="c", subcore_axis_name="s"),
    out_type=jax.ShapeDtypeStruct((NC, N_PAD, HALF), jnp.float32),
    scratch_types=[
        pltpu.VMEM((_EPP,), jnp.int32),
        pltpu.VMEM((_EPP,), jnp.int32),
        pltpu.VMEM((_EPP,), jnp.int32),
        pltpu.VMEM((_EPP,), jnp.int32),
        pltpu.VMEM((K,), jnp.int32),
        pltpu.VMEM((K,), jnp.int32),
        pltpu.VMEM((K, HALF), jnp.float32),
        pltpu.VMEM((K, HALF), jnp.float32),
        pltpu.VMEM_SHARED((N_PAD, HALF), jnp.float32),
        pltpu.SemaphoreType.DMA,
        pltpu.SemaphoreType.DMA,
        pltpu.SemaphoreType.DMA,
        pltpu.SemaphoreType.DMA,
        pltpu.SemaphoreType.DMA,
    ],
)
def _agg_kernel(hp_hbm, src_hbm, dst_hbm, zeros_hbm, out_hbm,
                src_a, src_b, dst_a, dst_b, dstv_a, dstv_b,
                rows_a, rows_b, acc_sh,
                sem_a, sem_b, sem_sa, sem_sb, sem_i):
    c = lax.axis_index("c")
    s = lax.axis_index("s")

    base = s * _EPT_AGG
    coff = c * N

    def adjust(src_buf):
        def fadj(i, carry):
            sl = pl.ds(i * 16, 16)
            src_buf[sl] = src_buf[sl] + coff
            return carry
        lax.fori_loop(0, _EPP // 16, fadj, 0)

    pltpu.sync_copy(src_hbm.at[pl.ds(base, _EPP)], src_a)
    pltpu.sync_copy(dst_hbm.at[pl.ds(base, _EPP)], dst_a)
    adjust(src_a)

    pltpu.sync_copy(zeros_hbm.at[pl.ds(0, K)], rows_a)
    for z in range(ROWS_PER_TILE // K):
        pltpu.sync_copy(
            rows_a, acc_sh.at[pl.ds(s * ROWS_PER_TILE + z * K, K)]
        )
    plsc.subcore_barrier()

    def g_start(src_buf, i, rows, sem):
        pltpu.make_async_copy(hp_hbm.at[src_buf.at[pl.ds(i * K, K)]],
                              rows, sem).start()

    def g_wait(src_buf, i, rows, sem):
        pltpu.make_async_copy(hp_hbm.at[src_buf.at[pl.ds(i * K, K)]],
                              rows, sem).wait()

    def idx_copy(dst_buf, i, dv):
        for v in range(K // 16):
            dv[pl.ds(v * 16, 16)] = dst_buf[pl.ds(i * K + v * 16, 16)]

    def s_start(rows, dv, sem):
        pltpu.async_copy(rows, acc_sh.at[dv], sem, add=True)

    def s_wait(rows, dv, sem):
        pltpu.make_async_copy(rows, acc_sh.at[dv], sem).wait()

    bufs = [(src_a, dst_a), (src_b, dst_b)]
    for p in range(_NPHASE):
        src_c, dst_c = bufs[p % 2]
        src_n, dst_n = bufs[(p + 1) % 2]
        if p + 1 < _NPHASE:
            noff = base + (p + 1) * _EPP
            pltpu.make_async_copy(
                src_hbm.at[pl.ds(noff, _EPP)], src_n, sem_i).start()
            pltpu.make_async_copy(
                dst_hbm.at[pl.ds(noff, _EPP)], dst_n, sem_i).start()

        g_start(src_c, 0, rows_a, sem_a)
        g_start(src_c, 1, rows_b, sem_b)

        def body(j, carry):
            i0 = 2 * j
            i1 = i0 + 1
            g_wait(src_c, i0, rows_a, sem_a)
            idx_copy(dst_c, i0, dstv_a)
            pltpu.sync_copy(rows_a, acc_sh.at[dstv_a], add=True)

            @pl.when(j < _NHP - 1)
            def _prea():
                g_start(src_c, i0 + 2, rows_a, sem_a)

            g_wait(src_c, i1, rows_b, sem_b)
            idx_copy(dst_c, i1, dstv_b)
            pltpu.sync_copy(rows_b, acc_sh.at[dstv_b], add=True)

            @pl.when(j < _NHP - 1)
            def _preb():
                g_start(src_c, i1 + 2, rows_b, sem_b)

            return carry

        lax.fori_loop(0, _NHP, body, 0)

        if p + 1 < _NPHASE:
            pltpu.make_async_copy(
                src_hbm.at[pl.ds(noff, _EPP)], src_n, sem_i).wait()
            pltpu.make_async_copy(
                dst_hbm.at[pl.ds(noff, _EPP)], dst_n, sem_i).wait()
            adjust(src_n)

    plsc.subcore_barrier()

    for z in range(ROWS_PER_TILE // K):
        zsl = pl.ds(s * ROWS_PER_TILE + z * K, K)
        pltpu.sync_copy(acc_sh.at[zsl], rows_a)
        pltpu.sync_copy(rows_a, out_hbm.at[c, zsl])


def _tc1_body(x_ref, w1_ref, degp_ref, hp_ref):
    deg = degp_ref[0] + degp_ref[1] + 1.0
    dis = lax.rsqrt(deg)
    h = jnp.dot(x_ref[...], w1_ref[...], preferred_element_type=jnp.float32)
    hp = h * dis
    hp_ref[0] = hp[:, :HALF]
    hp_ref[1] = hp[:, HALF:]


_tc1 = pl.pallas_call(
    _tc1_body,
    grid=(NBLK,),
    in_specs=[
        pl.BlockSpec((BN, F_IN), lambda i: (i, 0)),
        pl.BlockSpec((F_IN, H), lambda i: (0, 0)),
        pl.BlockSpec((NC, BN, 1), lambda i: (0, i, 0)),
    ],
    out_specs=pl.BlockSpec((NC, BN, HALF), lambda i: (0, i, 0)),
    out_shape=jax.ShapeDtypeStruct((NC, N, HALF), jnp.float32),
)


def _tc2_body(agg_ref, hp_ref, degp_ref, b1_ref, w2_ref, hq_ref):
    deg = degp_ref[0] + degp_ref[1] + 1.0
    dis = lax.rsqrt(deg)
    b1 = b1_ref[...]
    h1a = _leaky(dis * (agg_ref[0] + hp_ref[0]) + b1[:, :HALF])
    h1b = _leaky(dis * (agg_ref[1] + hp_ref[1]) + b1[:, HALF:])
    w2 = w2_ref[...]
    h2 = (jnp.dot(h1a, w2[:HALF, :], preferred_element_type=jnp.float32)
          + jnp.dot(h1b, w2[HALF:, :], preferred_element_type=jnp.float32))
    hq = h2 * dis
    hq_ref[0] = hq[:, :HALF]
    hq_ref[1] = hq[:, HALF:]


_tc2 = pl.pallas_call(
    _tc2_body,
    grid=(NBLK,),
    in_specs=[
        pl.BlockSpec((NC, BN, HALF), lambda i: (0, i, 0)),
        pl.BlockSpec((NC, BN, HALF), lambda i: (0, i, 0)),
        pl.BlockSpec((NC, BN, 1), lambda i: (0, i, 0)),
        pl.BlockSpec((1, H), lambda i: (0, 0)),
        pl.BlockSpec((H, H), lambda i: (0, 0)),
    ],
    out_specs=pl.BlockSpec((NC, BN, HALF), lambda i: (0, i, 0)),
    out_shape=jax.ShapeDtypeStruct((NC, N, HALF), jnp.float32),
)


def _tc3_body(agg_ref, hq_ref, degp_ref, b2_ref,
              batch_ref, wfc_ref, bfc_ref, out_ref, sums, cnts):
    i = pl.program_id(0)

    @pl.when(i == 0)
    def _init():
        sums[...] = jnp.zeros_like(sums)
        cnts[...] = jnp.zeros_like(cnts)

    deg = degp_ref[0] + degp_ref[1] + 1.0
    dis = lax.rsqrt(deg)
    b2 = b2_ref[...]
    h2a = _leaky(dis * (agg_ref[0] + hq_ref[0]) + b2[:, :HALF])
    h2b = _leaky(dis * (agg_ref[1] + hq_ref[1]) + b2[:, HALF:])
    h2 = jnp.concatenate([h2a, h2b], axis=1)

    bb = batch_ref[...]
    mgT = (bb == lax.broadcasted_iota(jnp.int32, (BN, G), 1)).astype(
        jnp.float32)
    sums[...] += lax.dot_general(
        mgT, h2, (((0,), (0,)), ((), ())), preferred_element_type=jnp.float32)
    cnts[...] += lax.dot_general(
        mgT, jnp.ones((BN, 128), jnp.float32), (((0,), (0,)), ((), ())),
        preferred_element_type=jnp.float32)

    @pl.when(i == NBLK - 1)
    def _fin():
        cnt = jnp.maximum(cnts[:, 0:1], 1.0)
        pooled = sums[...] / cnt
        z = jnp.dot(pooled, wfc_ref[...], preferred_element_type=jnp.float32)
        zb = z + bfc_ref[...]
        out_ref[...] = 1.0 / (1.0 + jnp.exp(-zb))


_tc3 = pl.pallas_call(
    _tc3_body,
    grid=(NBLK,),
    in_specs=[
        pl.BlockSpec((NC, BN, HALF), lambda i: (0, i, 0)),
        pl.BlockSpec((NC, BN, HALF), lambda i: (0, i, 0)),
        pl.BlockSpec((NC, BN, 1), lambda i: (0, i, 0)),
        pl.BlockSpec((1, H), lambda i: (0, 0)),
        pl.BlockSpec((BN, 1), lambda i: (i, 0)),
        pl.BlockSpec((H, 128), lambda i: (0, 0)),
        pl.BlockSpec((1, 128), lambda i: (0, 0)),
    ],
    out_specs=pl.BlockSpec((G, 128), lambda i: (0, 0)),
    out_shape=jax.ShapeDtypeStruct((G, 128), jnp.float32),
    scratch_shapes=[
        pltpu.VMEM((G, H), jnp.float32),
        pltpu.VMEM((G, 128), jnp.float32),
    ],
)


def kernel(x, edge_index, batch, W1, b1, W2, b2, Wfc, bfc):
    npad = E_PAD - E
    pad_iota = jnp.arange(npad, dtype=jnp.int32)
    src = jnp.concatenate([edge_index[0], pad_iota % N])
    dst = jnp.concatenate([edge_index[1], N + pad_iota % (N_PAD - N)])

    onesk = jnp.ones((K, HALF), jnp.float32)
    zerosk = jnp.zeros((ZROWS, HALF), jnp.float32)
    degp = _deg_kernel(dst, onesk, zerosk)[:, :, 0:1]
    hp = _tc1(x, W1, degp)
    agg = _agg_kernel(hp.reshape(NC * N, HALF), src, dst, zerosk)
    hq = _tc2(agg, hp, degp, b1.reshape(1, H), W2)
    aggb = _agg_kernel(hq.reshape(NC * N, HALF), src, dst, zerosk)

    wfc_pad = jnp.pad(Wfc, ((0, 0), (0, 127)))
    bfc_pad = jnp.broadcast_to(bfc.reshape(1, 1), (1, 128))
    out = _tc3(aggb, hq, degp, b2.reshape(1, H),
               batch.reshape(N, 1), wfc_pad, bfc_pad)
    return out[:, 0]
